# baseline (speedup 1.0000x reference)
import functools

import jax
import jax.numpy as jnp
from jax import lax
from jax.experimental import pallas as pl
from jax.experimental.pallas import tpu as pltpu
from jax.experimental.pallas import tpu_sc as plsc

B, L, D = 16, 4096, 300
BLK = 8
CT = 128
CB = CT // BLK
DPAD = 304
NJ = 18
NSUB = 16
NW = 2 * NSUB

_mesh = plsc.VectorSubcoreMesh(core_axis_name="c", subcore_axis_name="s")


@functools.partial(
    pl.kernel,
    mesh=_mesh,
    out_type=jax.ShapeDtypeStruct((NW, B, DPAD), jnp.float32),
    scratch_types=[
        pltpu.VMEM((CT, D), jnp.float32),
        pltpu.VMEM((CT, D), jnp.float32),
        pltpu.VMEM((B, DPAD), jnp.float32),
        pltpu.VMEM((B,), jnp.int32),
        pltpu.VMEM((B,), jnp.int32),
        pltpu.SemaphoreType.DMA,
        pltpu.SemaphoreType.DMA,
    ],
    compiler_params=pltpu.CompilerParams(needs_layout_passes=False),
)
def _awe_pool(x_hbm, len_hbm, part_hbm, bufa, bufb, prows, lens_v, gcnt_v,
              sema, semb):
    sid = lax.axis_index("s")
    cid = lax.axis_index("c")
    wid = cid * NSUB + sid
    zero16 = jnp.zeros((16,), jnp.float32)
    lane = lax.iota(jnp.int32, 16)

    pltpu.sync_copy(len_hbm, lens_v)
    lens_vec = lens_v[...]
    gvec = (lens_vec + (BLK - 1)) // BLK
    gcnt_v[...] = gvec

    def vext(ref, gidx):
        return plsc.load_gather(ref, [jnp.full((16,), gidx, jnp.int32)])[0]

    def add_token(buf, t, accs):
        out = [accs[j] + buf[t, pl.ds(16 * j, 16)] for j in range(NJ)]
        v = buf[t, pl.ds(D - 16, 16)]
        out.append(accs[NJ] + jnp.where(lane >= 4, v, 0.0))
        return tuple(out)

    def acc_chunk(buf, accs):
        def q16(q, accs):
            t = q * 16
            for u in range(16):
                accs = add_token(buf, t + u, accs)
            return accs

        return lax.fori_loop(0, CT // 16, q16, accs)

    total = jnp.int32(0)
    for s in range(B):
        total = total + vext(gcnt_v, s)
    quota = (total + NW - 1) // NW
    lo = jnp.minimum(wid * quota, total)
    hi = jnp.minimum(lo + quota, total)

    def sent_body(s, base):
        gcount = vext(gcnt_v, s)
        a = jnp.clip(lo - base, 0, gcount)
        b = jnp.clip(hi - base, 0, gcount)
        lenT = vext(lens_v, s)

        def start(buf, sem, i, a=a, s=s):
            t0 = (a + i * CB) * BLK
            pltpu.async_copy(x_hbm.at[s, pl.ds(t0, CT), :], buf, sem)

        def wait(buf, sem, s=s):
            pltpu.make_async_copy(x_hbm.at[s, pl.ds(0, CT), :], buf, sem).wait()

        def compute(a=a, b=b, lenT=lenT, s=s, start=start, wait=wait):
            accs = tuple(zero16 for _ in range(NJ + 1))
            ragged = (b * BLK > lenT).astype(jnp.int32)
            nfull = (b - a - ragged) // CB
            ntail = (b - a) - nfull * CB

            @pl.when(nfull > 0)
            def _():
                start(bufa, sema, 0)

            @pl.when(nfull > 1)
            def _():
                start(bufb, semb, 1)

            def pair_body(p, accs):
                i0 = 2 * p
                wait(bufa, sema)

                @pl.when(i0 + 2 < nfull)
                def _():
                    start(bufa, sema, i0 + 2)

                accs = acc_chunk(bufa, accs)
                wait(bufb, semb)

                @pl.when(i0 + 3 < nfull)
                def _():
                    start(bufb, semb, i0 + 3)

                return acc_chunk(bufb, accs)

            accs = lax.fori_loop(0, nfull // 2, pair_body, accs)

            def odd_chunk(accs):
                wait(bufa, sema)
                return acc_chunk(bufa, accs)

            accs = lax.cond(nfull % 2 == 1, odd_chunk, lambda x: x, accs)

            def tail_body(i, accs):
                blk = a + nfull * CB + i
                t0 = blk * BLK
                pltpu.sync_copy(
                    x_hbm.at[s, pl.ds(t0, BLK), :], bufa.at[pl.ds(0, BLK), :]
                )
                nv = jnp.clip(lenT - t0, 0, BLK)

                def tb(t, accs):
                    return add_token(bufa, t, accs)

                return lax.fori_loop(0, nv, tb, accs)

            return lax.fori_loop(0, ntail, tail_body, accs)

        def empty():
            return tuple(zero16 for _ in range(NJ + 1))

        accs = lax.cond(b > a, compute, empty)

        for j in range(NJ + 1):
            prows[s, pl.ds(16 * j, 16)] = accs[j]
        return base + gcount

    lax.fori_loop(0, B, sent_body, jnp.int32(0))

    pltpu.sync_copy(prows, part_hbm.at[wid])


def _combine_body(part_ref, len_ref, out_ref):
    acc = part_ref[0]
    for w in range(1, NW):
        acc = acc + part_ref[w]
    head = acc[:, : NJ * 16]
    tail = acc[:, NJ * 16 + 4:]
    pad = jnp.zeros((B, 4), jnp.float32)
    full = jnp.concatenate([head, tail, pad], axis=1)
    out_ref[...] = full / len_ref[...]


_combine = pl.pallas_call(
    _combine_body,
    out_shape=jax.ShapeDtypeStruct((B, DPAD), jnp.float32),
)


def kernel(sentences, sentence_lengths):
    parts = _awe_pool(sentences, sentence_lengths)
    lens_f = sentence_lengths.astype(jnp.float32).reshape(B, 1)
    out = _combine(parts, lens_f)
    return out[:, :D]


# speedup vs baseline: 1.0896x; 1.0896x over previous
import functools

import jax
import jax.numpy as jnp
from jax import lax
from jax.experimental import pallas as pl
from jax.experimental.pallas import tpu as pltpu
from jax.experimental.pallas import tpu_sc as plsc

B, L, D = 16, 4096, 300
BLK = 8
CT = 128
CB = CT // BLK
DPAD = 304
NJ = 18
NSUB = 16
NW = 2 * NSUB

_mesh = plsc.VectorSubcoreMesh(core_axis_name="c", subcore_axis_name="s")


@functools.partial(
    pl.kernel,
    mesh=_mesh,
    out_type=jax.ShapeDtypeStruct((NW, B, DPAD), jnp.float32),
    scratch_types=[
        pltpu.VMEM((CT, D), jnp.float32),
        pltpu.VMEM((CT, D), jnp.float32),
        pltpu.VMEM((B, DPAD), jnp.float32),
        pltpu.VMEM((B,), jnp.int32),
        pltpu.VMEM((B,), jnp.int32),
        pltpu.SemaphoreType.DMA,
        pltpu.SemaphoreType.DMA,
    ],
    compiler_params=pltpu.CompilerParams(needs_layout_passes=False),
)
def _awe_pool(x_hbm, len_hbm, part_hbm, bufa, bufb, prows, lens_v, gcnt_v,
              sema, semb):
    sid = lax.axis_index("s")
    cid = lax.axis_index("c")
    wid = cid * NSUB + sid
    zero16 = jnp.zeros((16,), jnp.float32)
    lane = lax.iota(jnp.int32, 16)

    pltpu.sync_copy(len_hbm, lens_v)
    lens_vec = lens_v[...]
    gvec = (lens_vec + (BLK - 1)) // BLK
    gcnt_v[...] = gvec

    def vext(ref, gidx):
        return plsc.load_gather(ref, [jnp.full((16,), gidx, jnp.int32)])[0]

    def add_token(buf, t, accs):
        out = [accs[j] + buf[t, pl.ds(16 * j, 16)] for j in range(NJ)]
        v = buf[t, pl.ds(D - 16, 16)]
        out.append(accs[NJ] + jnp.where(lane >= 4, v, 0.0))
        return tuple(out)

    def acc_chunk(buf, accs):
        def q4(q, accs):
            t = q * 4
            for u in range(4):
                accs = add_token(buf, t + u, accs)
            return accs

        return lax.fori_loop(0, CT // 4, q4, accs)

    total = jnp.int32(0)
    for s in range(B):
        total = total + vext(gcnt_v, s)
    quota = (total + NW - 1) // NW
    lo = jnp.minimum(wid * quota, total)
    hi = jnp.minimum(lo + quota, total)

    def sent_body(s, base):
        gcount = vext(gcnt_v, s)
        a = jnp.clip(lo - base, 0, gcount)
        b = jnp.clip(hi - base, 0, gcount)
        lenT = vext(lens_v, s)

        def start(buf, sem, i, a=a, s=s):
            t0 = (a + i * CB) * BLK
            pltpu.async_copy(x_hbm.at[s, pl.ds(t0, CT), :], buf, sem)

        def wait(buf, sem, s=s):
            pltpu.make_async_copy(x_hbm.at[s, pl.ds(0, CT), :], buf, sem).wait()

        def compute(a=a, b=b, lenT=lenT, s=s, start=start, wait=wait):
            accs = tuple(zero16 for _ in range(NJ + 1))
            ragged = (b * BLK > lenT).astype(jnp.int32)
            nfull = (b - a - ragged) // CB
            ntail = (b - a) - nfull * CB

            @pl.when(nfull > 0)
            def _():
                start(bufa, sema, 0)

            @pl.when(nfull > 1)
            def _():
                start(bufb, semb, 1)

            def pair_body(p, accs):
                i0 = 2 * p
                wait(bufa, sema)

                @pl.when(i0 + 2 < nfull)
                def _():
                    start(bufa, sema, i0 + 2)

                accs = acc_chunk(bufa, accs)
                wait(bufb, semb)

                @pl.when(i0 + 3 < nfull)
                def _():
                    start(bufb, semb, i0 + 3)

                return acc_chunk(bufb, accs)

            accs = lax.fori_loop(0, nfull // 2, pair_body, accs)

            def odd_chunk(accs):
                wait(bufa, sema)
                return acc_chunk(bufa, accs)

            accs = lax.cond(nfull % 2 == 1, odd_chunk, lambda x: x, accs)

            def tail_body(i, accs):
                blk = a + nfull * CB + i
                t0 = blk * BLK
                pltpu.sync_copy(
                    x_hbm.at[s, pl.ds(t0, BLK), :], bufa.at[pl.ds(0, BLK), :]
                )
                nv = jnp.clip(lenT - t0, 0, BLK)

                def tb(t, accs):
                    return add_token(bufa, t, accs)

                return lax.fori_loop(0, nv, tb, accs)

            return lax.fori_loop(0, ntail, tail_body, accs)

        def empty():
            return tuple(zero16 for _ in range(NJ + 1))

        accs = lax.cond(b > a, compute, empty)

        for j in range(NJ + 1):
            prows[s, pl.ds(16 * j, 16)] = accs[j]
        return base + gcount

    lax.fori_loop(0, B, sent_body, jnp.int32(0))

    pltpu.sync_copy(prows, part_hbm.at[wid])


def _combine_body(part_ref, len_ref, out_ref):
    acc = part_ref[0]
    for w in range(1, NW):
        acc = acc + part_ref[w]
    head = acc[:, : NJ * 16]
    tail = acc[:, NJ * 16 + 4:]
    pad = jnp.zeros((B, 4), jnp.float32)
    full = jnp.concatenate([head, tail, pad], axis=1)
    out_ref[...] = full / len_ref[...]


_combine = pl.pallas_call(
    _combine_body,
    out_shape=jax.ShapeDtypeStruct((B, DPAD), jnp.float32),
)


def kernel(sentences, sentence_lengths):
    parts = _awe_pool(sentences, sentence_lengths)
    lens_f = sentence_lengths.astype(jnp.float32).reshape(B, 1)
    out = _combine(parts, lens_f)
    return out[:, :D]
